# baseline (speedup 1.0000x reference)
import functools

import jax
import jax.numpy as jnp
from jax import lax
from jax.experimental import pallas as pl
from jax.experimental.pallas import tpu as pltpu
from jax.experimental.pallas import tpu_sc as plsc

_NC = 2
_NS = 16
_C = 40
_LANES = 16


def _matmul_body(x_ref, w_ref, o_ref):
    o_ref[...] = jnp.dot(x_ref[...], w_ref[...],
                         preferred_element_type=jnp.float32)


def _dense_transform(x, w):
    n, d = x.shape
    u = w.shape[1]
    bm = 1000
    return pl.pallas_call(
        _matmul_body,
        grid=(n // bm,),
        in_specs=[
            pl.BlockSpec((bm, d), lambda i: (i, 0)),
            pl.BlockSpec((d, u), lambda i: (0, 0)),
        ],
        out_specs=pl.BlockSpec((bm, u), lambda i: (i, 0)),
        out_shape=jax.ShapeDtypeStruct((n, u), jnp.float32),
    )(x, w)


def _combine_body(a_ref, b_ref, o_ref):
    o_ref[...] = jnp.maximum(a_ref[...] + b_ref[...], 0.0)


def _combine_relu(p0, p1):
    n, d = p0.shape
    bm = 1000
    return pl.pallas_call(
        _combine_body,
        grid=(n // bm,),
        in_specs=[
            pl.BlockSpec((bm, d), lambda i: (i, 0)),
            pl.BlockSpec((bm, d), lambda i: (i, 0)),
        ],
        out_specs=pl.BlockSpec((bm, d), lambda i: (i, 0)),
        out_shape=jax.ShapeDtypeStruct((n, d), jnp.float32),
    )(p0, p1)


_NBUF = 5


def _edge_body(npad, d, e, h_hbm, src_hbm, dst_hbm, adj_hbm, zeros_hbm,
               out_hbm, srcb, dstb, adjb, rows_v, acc_sh, *sems):
    isem = sems[:_NBUF]
    gsem = sems[_NBUF:2 * _NBUF]
    ssem = sems[2 * _NBUF:]
    c = lax.axis_index("c")
    s = lax.axis_index("s")
    nw = _NC * _NS
    e_per = e // nw
    k_chunks = e_per // _C
    rows_per_tile = npad // _NS
    dgroups = d // _LANES
    w = c * _NS + s
    ebase = w * e_per

    pltpu.sync_copy(zeros_hbm, acc_sh.at[pl.ds(s * rows_per_tile,
                                               rows_per_tile)])
    plsc.subcore_barrier()

    def issue_idx(k, b):
        o = ebase + k * _C
        pltpu.async_copy(src_hbm.at[pl.ds(o, _C)], srcb.at[b], isem[b])
        pltpu.async_copy(dst_hbm.at[pl.ds(o, _C)], dstb.at[b], isem[b])
        pltpu.async_copy(adj_hbm.at[pl.ds(o, _C)], adjb.at[b], isem[b])

    def wait_idx(k, b):
        o = ebase + k * _C
        pltpu.make_async_copy(src_hbm.at[pl.ds(o, _C)], srcb.at[b],
                              isem[b]).wait()
        pltpu.make_async_copy(dst_hbm.at[pl.ds(o, _C)], dstb.at[b],
                              isem[b]).wait()
        pltpu.make_async_copy(adj_hbm.at[pl.ds(o, _C)], adjb.at[b],
                              isem[b]).wait()

    def issue_gather(b):
        pltpu.async_copy(h_hbm.at[srcb.at[b]], rows_v.at[b], gsem[b])

    def wait_gather(b):
        pltpu.make_async_copy(h_hbm.at[srcb.at[b]], rows_v.at[b],
                              gsem[b]).wait()

    def issue_scatter(b):
        pltpu.async_copy(rows_v.at[b], acc_sh.at[dstb.at[b]], ssem[b],
                         add=True)

    def wait_scatter(b):
        pltpu.make_async_copy(rows_v.at[b], acc_sh.at[dstb.at[b]],
                              ssem[b]).wait()

    def do_step(k, i, wait_sc=True, do_idx=True, do_g=True):
        wait_gather(i)
        if wait_sc:
            wait_scatter((i + 2) % _NBUF)
        if do_idx:
            issue_idx(k + 2, (i + 2) % _NBUF)
        if do_g:
            wait_idx(k + 1, (i + 1) % _NBUF)
            issue_gather((i + 1) % _NBUF)
        rv = rows_v.at[i]
        av = adjb.at[i]

        def edge(ei, carry2):
            idx = jnp.full((_LANES,), 0, jnp.int32) + ei
            scale = plsc.load_gather(av, [idx])
            for g in range(dgroups):
                sl = pl.ds(g * _LANES, _LANES)
                rv[ei, sl] = rv[ei, sl] * scale
            return carry2

        lax.fori_loop(0, _C, edge, 0, unroll=4)
        issue_scatter(i)

    n_blocks = k_chunks // _NBUF
    issue_idx(0, 0)
    issue_idx(1, 1)
    wait_idx(0, 0)
    issue_gather(0)
    for i in range(_NBUF):
        do_step(i, i, wait_sc=(i >= 3))

    def block(j, carry):
        for i in range(_NBUF):
            do_step(j * _NBUF + i, i)
        return carry

    lax.fori_loop(1, n_blocks - 1, block, 0)

    for i in range(_NBUF):
        k = (n_blocks - 1) * _NBUF + i
        do_step(k, i, do_idx=(i < 3), do_g=(i < _NBUF - 1))

    for i in range(2, _NBUF):
        wait_scatter(i)

    plsc.subcore_barrier()

    pltpu.sync_copy(acc_sh.at[pl.ds(s * rows_per_tile, rows_per_tile)],
                    out_hbm.at[pl.ds(c * npad + s * rows_per_tile,
                                     rows_per_tile)])


def _edge_aggregate(h, src, dst, adj):
    n, d = h.shape
    e = src.shape[0]
    npad = ((n + 8 * _NS - 1) // (8 * _NS)) * (8 * _NS)
    rows_per_tile = npad // _NS
    zeros = jnp.zeros((rows_per_tile, d), jnp.float32)
    mesh = plsc.VectorSubcoreMesh(core_axis_name="c", subcore_axis_name="s",
                                  num_cores=_NC, num_subcores=_NS)
    body = functools.partial(_edge_body, npad, d, e)
    partials = pl.kernel(
        body,
        out_type=jax.ShapeDtypeStruct((_NC * npad, d), jnp.float32),
        mesh=mesh,
        compiler_params=pltpu.CompilerParams(needs_layout_passes=False),
        scratch_types=[
            pltpu.VMEM((_NBUF, _C), jnp.int32),
            pltpu.VMEM((_NBUF, _C), jnp.int32),
            pltpu.VMEM((_NBUF, _C), jnp.float32),
            pltpu.VMEM((_NBUF, _C, d), jnp.float32),
            pltpu.VMEM_SHARED((npad, d), jnp.float32),
        ] + [pltpu.SemaphoreType.DMA] * (3 * _NBUF),
    )(h, src, dst, adj, zeros)
    return partials, npad


def kernel(x, edge_index, adj_values, kernel):
    n = x.shape[0]
    h = _dense_transform(x, kernel)
    src = edge_index[0].astype(jnp.int32)
    dst = edge_index[1].astype(jnp.int32)
    partials, npad = _edge_aggregate(h, src, dst, adj_values)
    return _combine_relu(partials[:n], partials[npad:npad + n])


# speedup vs baseline: 1.2296x; 1.2296x over previous
import functools

import jax
import jax.numpy as jnp
from jax import lax
from jax.experimental import pallas as pl
from jax.experimental.pallas import tpu as pltpu
from jax.experimental.pallas import tpu_sc as plsc

_NC = 2
_NS = 16
_C = 80
_NBUF = 3
_LANES = 16
_ZROWS = 632


def _matmul_body(x_ref, w_ref, o_ref):
    o_ref[...] = jnp.dot(x_ref[...], w_ref[...],
                         preferred_element_type=jnp.float32)


def _dense_transform(x, w):
    n, d = x.shape
    u = w.shape[1]
    bm = 1000
    return pl.pallas_call(
        _matmul_body,
        grid=(n // bm,),
        in_specs=[
            pl.BlockSpec((bm, d), lambda i: (i, 0)),
            pl.BlockSpec((d, u), lambda i: (0, 0)),
        ],
        out_specs=pl.BlockSpec((bm, u), lambda i: (i, 0)),
        out_shape=jax.ShapeDtypeStruct((n, u), jnp.float32),
    )(x, w)


def _combine_body(a_ref, b_ref, o_ref):
    o_ref[...] = jnp.maximum(a_ref[...] + b_ref[...], 0.0)


def _combine_relu(p0, p1):
    n, d = p0.shape
    bm = 1000
    return pl.pallas_call(
        _combine_body,
        grid=(n // bm,),
        in_specs=[
            pl.BlockSpec((bm, d), lambda i: (i, 0)),
            pl.BlockSpec((bm, d), lambda i: (i, 0)),
        ],
        out_specs=pl.BlockSpec((bm, d), lambda i: (i, 0)),
        out_shape=jax.ShapeDtypeStruct((n, d), jnp.float32),
    )(p0, p1)


def _edge_body(n, d, e, h_hbm, src_hbm, dst_hbm, adj_hbm, zeros_hbm,
               out_hbm, srcb, dstb, adjb, rows_v, acc_sh, *sems):
    isem = sems[:_NBUF]
    gsem = sems[_NBUF:2 * _NBUF]
    ssem = sems[2 * _NBUF:]
    c = lax.axis_index("c")
    s = lax.axis_index("s")
    nw = _NC * _NS
    e_per = e // nw
    k_chunks = e_per // _C
    rows_per_tile = n // _NS
    dgroups = d // _LANES
    w = c * _NS + s
    ebase = w * e_per
    zoff = pl.multiple_of(s * rows_per_tile - lax.rem(s, 8), 8)

    pltpu.sync_copy(zeros_hbm, acc_sh.at[pl.ds(zoff, _ZROWS)])
    plsc.subcore_barrier()

    def issue_idx(k, b):
        o = ebase + k * _C
        pltpu.async_copy(src_hbm.at[pl.ds(o, _C)], srcb.at[b], isem[b])
        pltpu.async_copy(dst_hbm.at[pl.ds(o, _C)], dstb.at[b], isem[b])
        pltpu.async_copy(adj_hbm.at[pl.ds(o, _C)], adjb.at[b], isem[b])

    def wait_idx(k, b):
        o = ebase + k * _C
        pltpu.make_async_copy(src_hbm.at[pl.ds(o, _C)], srcb.at[b],
                              isem[b]).wait()
        pltpu.make_async_copy(dst_hbm.at[pl.ds(o, _C)], dstb.at[b],
                              isem[b]).wait()
        pltpu.make_async_copy(adj_hbm.at[pl.ds(o, _C)], adjb.at[b],
                              isem[b]).wait()

    def issue_gather(b):
        pltpu.async_copy(h_hbm.at[srcb.at[b]], rows_v.at[b], gsem[b])

    def wait_gather(b):
        pltpu.make_async_copy(h_hbm.at[srcb.at[b]], rows_v.at[b],
                              gsem[b]).wait()

    def issue_scatter(b):
        pltpu.async_copy(rows_v.at[b], acc_sh.at[dstb.at[b]], ssem[b],
                         add=True)

    def wait_scatter(b):
        pltpu.make_async_copy(rows_v.at[b], acc_sh.at[dstb.at[b]],
                              ssem[b]).wait()

    def do_step(k, i, wait_sc=True, do_idx=True, do_g=True):
        wait_gather(i)
        if do_g:
            wait_idx(k + 1, (i + 1) % _NBUF)
            issue_gather((i + 1) % _NBUF)
        rv = rows_v.at[i]
        av = adjb.at[i]

        def edge(ei, carry2):
            idx = jnp.full((_LANES,), 0, jnp.int32) + ei
            scale = plsc.load_gather(av, [idx])
            for g in range(dgroups):
                sl = pl.ds(g * _LANES, _LANES)
                rv[ei, sl] = rv[ei, sl] * scale
            return carry2

        lax.fori_loop(0, _C, edge, 0)
        if wait_sc:
            wait_scatter((i + 2) % _NBUF)
        if do_idx:
            issue_idx(k + 2, (i + 2) % _NBUF)
        issue_scatter(i)

    issue_idx(0, 0)
    issue_idx(1, 1)
    wait_idx(0, 0)
    issue_gather(0)
    do_step(0, 0, wait_sc=False)

    n_steady = (k_chunks - 5) // _NBUF

    def block(j, carry):
        k0 = _NBUF * j + 1
        for i in range(_NBUF):
            do_step(k0 + i, (1 + i) % _NBUF)
        return carry

    lax.fori_loop(0, n_steady, block, 0)

    for t in range(4):
        k = k_chunks - 4 + t
        do_step(k, k % _NBUF, do_idx=(t < 2), do_g=(t < 3))

    wait_scatter((k_chunks - 1) % _NBUF)

    plsc.subcore_barrier()

    pltpu.sync_copy(acc_sh.at[pl.ds(zoff, _ZROWS)],
                    out_hbm.at[pl.ds(c * n + zoff, _ZROWS)])


def _edge_aggregate(h, src, dst, adj):
    n, d = h.shape
    e = src.shape[0]
    zeros = jnp.zeros((_ZROWS, d), jnp.float32)
    mesh = plsc.VectorSubcoreMesh(core_axis_name="c", subcore_axis_name="s",
                                  num_cores=_NC, num_subcores=_NS)
    body = functools.partial(_edge_body, n, d, e)
    partials = pl.kernel(
        body,
        out_type=jax.ShapeDtypeStruct((_NC * n, d), jnp.float32),
        mesh=mesh,
        compiler_params=pltpu.CompilerParams(needs_layout_passes=False),
        scratch_types=[
            pltpu.VMEM((_NBUF, _C), jnp.int32),
            pltpu.VMEM((_NBUF, _C), jnp.int32),
            pltpu.VMEM((_NBUF, _C), jnp.float32),
            pltpu.VMEM((_NBUF, _C, d), jnp.float32),
            pltpu.VMEM_SHARED((n, d), jnp.float32),
        ] + [pltpu.SemaphoreType.DMA] * (3 * _NBUF),
    )(h, src, dst, adj, zeros)
    return partials


def kernel(x, edge_index, adj_values, kernel):
    n = x.shape[0]
    h = _dense_transform(x, kernel)
    src = edge_index[0].astype(jnp.int32)
    dst = edge_index[1].astype(jnp.int32)
    partials = _edge_aggregate(h, src, dst, adj_values)
    return _combine_relu(partials[:n], partials[n:])


# speedup vs baseline: 1.2667x; 1.0302x over previous
import functools

import jax
import jax.numpy as jnp
from jax import lax
from jax.experimental import pallas as pl
from jax.experimental.pallas import tpu as pltpu
from jax.experimental.pallas import tpu_sc as plsc

_NC = 2
_NS = 16
_C = 80
_NBUF = 3
_LANES = 16
_ZROWS = 632


def _matmul_body(x_ref, w_ref, o_ref):
    o_ref[...] = jnp.dot(x_ref[...], w_ref[...],
                         preferred_element_type=jnp.float32)


def _dense_transform(x, w):
    n, d = x.shape
    u = w.shape[1]
    bm = 1000
    return pl.pallas_call(
        _matmul_body,
        grid=(n // bm,),
        in_specs=[
            pl.BlockSpec((bm, d), lambda i: (i, 0)),
            pl.BlockSpec((d, u), lambda i: (0, 0)),
        ],
        out_specs=pl.BlockSpec((bm, u), lambda i: (i, 0)),
        out_shape=jax.ShapeDtypeStruct((n, u), jnp.float32),
    )(x, w)


def _combine_body(a_ref, b_ref, o_ref):
    o_ref[...] = jnp.maximum(a_ref[...] + b_ref[...], 0.0)


def _combine_relu(partials):
    n2, d = partials.shape
    n = n2 // 2
    bm = 1000
    nb = n // bm
    return pl.pallas_call(
        _combine_body,
        grid=(nb,),
        in_specs=[
            pl.BlockSpec((bm, d), lambda i: (i, 0)),
            pl.BlockSpec((bm, d), lambda i, _nb=nb: (i + _nb, 0)),
        ],
        out_specs=pl.BlockSpec((bm, d), lambda i: (i, 0)),
        out_shape=jax.ShapeDtypeStruct((n, d), jnp.float32),
    )(partials, partials)


def _edge_body(n, d, e, h_hbm, src_hbm, dst_hbm, adj_hbm, zeros_hbm,
               out_hbm, srcb, dstb, adjb, rows_v, acc_sh, *sems):
    isem = sems[:_NBUF]
    gsem = sems[_NBUF:2 * _NBUF]
    ssem = sems[2 * _NBUF:]
    c = lax.axis_index("c")
    s = lax.axis_index("s")
    nw = _NC * _NS
    e_per = e // nw
    k_chunks = e_per // _C
    rows_per_tile = n // _NS
    dgroups = d // _LANES
    w = c * _NS + s
    ebase = w * e_per
    zoff = pl.multiple_of(s * rows_per_tile - lax.rem(s, 8), 8)

    pltpu.sync_copy(zeros_hbm, acc_sh.at[pl.ds(zoff, _ZROWS)])
    plsc.subcore_barrier()

    def issue_idx(k, b):
        o = ebase + k * _C
        pltpu.async_copy(src_hbm.at[pl.ds(o, _C)], srcb.at[b], isem[b])
        pltpu.async_copy(dst_hbm.at[pl.ds(o, _C)], dstb.at[b], isem[b])
        pltpu.async_copy(adj_hbm.at[pl.ds(o, _C)], adjb.at[b], isem[b])

    def wait_idx(k, b):
        o = ebase + k * _C
        pltpu.make_async_copy(src_hbm.at[pl.ds(o, _C)], srcb.at[b],
                              isem[b]).wait()
        pltpu.make_async_copy(dst_hbm.at[pl.ds(o, _C)], dstb.at[b],
                              isem[b]).wait()
        pltpu.make_async_copy(adj_hbm.at[pl.ds(o, _C)], adjb.at[b],
                              isem[b]).wait()

    def issue_gather(b):
        pltpu.async_copy(h_hbm.at[srcb.at[b]], rows_v.at[b], gsem[b])

    def wait_gather(b):
        pltpu.make_async_copy(h_hbm.at[srcb.at[b]], rows_v.at[b],
                              gsem[b]).wait()

    def issue_scatter(b):
        pltpu.async_copy(rows_v.at[b], acc_sh.at[dstb.at[b]], ssem[b],
                         add=True)

    def wait_scatter(b):
        pltpu.make_async_copy(rows_v.at[b], acc_sh.at[dstb.at[b]],
                              ssem[b]).wait()

    def do_step(k, i, wait_sc=True, do_idx=True, do_g=True):
        wait_gather(i)
        if do_g:
            wait_idx(k + 1, (i + 1) % _NBUF)
            issue_gather((i + 1) % _NBUF)
        rv = rows_v.at[i]
        av = adjb.at[i]

        def edge(ei, carry2):
            idx = jnp.full((_LANES,), 0, jnp.int32) + ei
            scale = plsc.load_gather(av, [idx])
            for g in range(dgroups):
                sl = pl.ds(g * _LANES, _LANES)
                rv[ei, sl] = rv[ei, sl] * scale
            return carry2

        lax.fori_loop(0, _C, edge, 0)
        if wait_sc:
            wait_scatter((i + 2) % _NBUF)
        if do_idx:
            issue_idx(k + 2, (i + 2) % _NBUF)
        issue_scatter(i)

    issue_idx(0, 0)
    issue_idx(1, 1)
    wait_idx(0, 0)
    issue_gather(0)
    do_step(0, 0, wait_sc=False)

    n_steady = (k_chunks - 5) // _NBUF

    def block(j, carry):
        k0 = _NBUF * j + 1
        for i in range(_NBUF):
            do_step(k0 + i, (1 + i) % _NBUF)
        return carry

    lax.fori_loop(0, n_steady, block, 0)

    for t in range(4):
        k = k_chunks - 4 + t
        do_step(k, k % _NBUF, do_idx=(t < 2), do_g=(t < 3))

    wait_scatter((k_chunks - 1) % _NBUF)

    plsc.subcore_barrier()

    pltpu.sync_copy(acc_sh.at[pl.ds(zoff, _ZROWS)],
                    out_hbm.at[pl.ds(c * n + zoff, _ZROWS)])


def _edge_aggregate(h, src, dst, adj):
    n, d = h.shape
    e = src.shape[0]
    zeros = jnp.zeros((_ZROWS, d), jnp.float32)
    mesh = plsc.VectorSubcoreMesh(core_axis_name="c", subcore_axis_name="s",
                                  num_cores=_NC, num_subcores=_NS)
    body = functools.partial(_edge_body, n, d, e)
    partials = pl.kernel(
        body,
        out_type=jax.ShapeDtypeStruct((_NC * n, d), jnp.float32),
        mesh=mesh,
        compiler_params=pltpu.CompilerParams(needs_layout_passes=False),
        scratch_types=[
            pltpu.VMEM((_NBUF, _C), jnp.int32),
            pltpu.VMEM((_NBUF, _C), jnp.int32),
            pltpu.VMEM((_NBUF, _C), jnp.float32),
            pltpu.VMEM((_NBUF, _C, d), jnp.float32),
            pltpu.VMEM_SHARED((n, d), jnp.float32),
        ] + [pltpu.SemaphoreType.DMA] * (3 * _NBUF),
    )(h, src, dst, adj, zeros)
    return partials


def kernel(x, edge_index, adj_values, kernel):
    n = x.shape[0]
    h = _dense_transform(x, kernel)
    src = edge_index[0].astype(jnp.int32)
    dst = edge_index[1].astype(jnp.int32)
    partials = _edge_aggregate(h, src, dst, adj_values)
    return _combine_relu(partials)


# speedup vs baseline: 1.5041x; 1.1874x over previous
import functools

import jax
import jax.numpy as jnp
from jax import lax
from jax.experimental import pallas as pl
from jax.experimental.pallas import tpu as pltpu
from jax.experimental.pallas import tpu_sc as plsc

_NC = 2
_NS = 16
_C = 80
_NG = 3
_NI = 4
_LANES = 16
_ZROWS = 632


def _matmul_body(x_ref, w_ref, o_ref):
    o_ref[...] = jnp.dot(x_ref[...], w_ref[...],
                         preferred_element_type=jnp.float32)


def _dense_transform(x, w):
    n, d = x.shape
    u = w.shape[1]
    bm = 1000
    return pl.pallas_call(
        _matmul_body,
        grid=(n // bm,),
        in_specs=[
            pl.BlockSpec((bm, d), lambda i: (i, 0)),
            pl.BlockSpec((d, u), lambda i: (0, 0)),
        ],
        out_specs=pl.BlockSpec((bm, u), lambda i: (i, 0)),
        out_shape=jax.ShapeDtypeStruct((n, u), jnp.float32),
    )(x, w)


def _combine_body(a_ref, b_ref, o_ref):
    o_ref[...] = jnp.maximum(a_ref[...] + b_ref[...], 0.0)


def _combine_relu(partials):
    n2, d = partials.shape
    n = n2 // 2
    bm = 1000
    nb = n // bm
    return pl.pallas_call(
        _combine_body,
        grid=(nb,),
        in_specs=[
            pl.BlockSpec((bm, d), lambda i: (i, 0)),
            pl.BlockSpec((bm, d), lambda i, _nb=nb: (i + _nb, 0)),
        ],
        out_specs=pl.BlockSpec((bm, d), lambda i: (i, 0)),
        out_shape=jax.ShapeDtypeStruct((n, d), jnp.float32),
    )(partials, partials)


def _edge_body(n, d, e, h_hbm, src_hbm, dst_hbm, adj_hbm, zeros_hbm,
               out_hbm, srcb, dstb, adjb, rows_v, acc_sh, *sems):
    isem = sems[:_NI]
    gsem = sems[_NI:_NI + _NG]
    ssem = sems[_NI + _NG:]
    c = lax.axis_index("c")
    s = lax.axis_index("s")
    nw = _NC * _NS
    e_per = e // nw
    k_chunks = e_per // _C
    rows_per_tile = n // _NS
    dgroups = d // _LANES
    w = c * _NS + s
    ebase = w * e_per
    zoff = pl.multiple_of(s * rows_per_tile - lax.rem(s, 8), 8)

    pltpu.sync_copy(zeros_hbm, acc_sh.at[pl.ds(zoff, _ZROWS)])
    plsc.subcore_barrier()

    def issue_idx(k, b):
        o = ebase + k * _C
        pltpu.async_copy(src_hbm.at[pl.ds(o, _C)], srcb.at[b], isem[b])
        pltpu.async_copy(dst_hbm.at[pl.ds(o, _C)], dstb.at[b], isem[b])
        pltpu.async_copy(adj_hbm.at[pl.ds(o, _C)], adjb.at[b], isem[b])

    def wait_idx(k, b):
        o = ebase + k * _C
        pltpu.make_async_copy(src_hbm.at[pl.ds(o, _C)], srcb.at[b],
                              isem[b]).wait()
        pltpu.make_async_copy(dst_hbm.at[pl.ds(o, _C)], dstb.at[b],
                              isem[b]).wait()
        pltpu.make_async_copy(adj_hbm.at[pl.ds(o, _C)], adjb.at[b],
                              isem[b]).wait()

    def issue_gather(bi, bg):
        pltpu.async_copy(h_hbm.at[srcb.at[bi]], rows_v.at[bg], gsem[bg])

    def wait_gather(bi, bg):
        pltpu.make_async_copy(h_hbm.at[srcb.at[bi]], rows_v.at[bg],
                              gsem[bg]).wait()

    def issue_scatter(bi, bg):
        pltpu.async_copy(rows_v.at[bg], acc_sh.at[dstb.at[bi]], ssem[bg],
                         add=True)

    def wait_scatter(bi, bg):
        pltpu.make_async_copy(rows_v.at[bg], acc_sh.at[dstb.at[bi]],
                              ssem[bg]).wait()

    def do_step(k, ib, g, wait_sc=True, do_idx=True, do_g2=True):
        wait_gather(ib, g)
        rv = rows_v.at[g]
        av = adjb.at[ib]

        def edge(ei, carry2):
            idx = jnp.full((_LANES,), 0, jnp.int32) + ei
            scale = plsc.load_gather(av, [idx])
            for gg in range(dgroups):
                sl = pl.ds(gg * _LANES, _LANES)
                rv[ei, sl] = rv[ei, sl] * scale
            return carry2

        lax.fori_loop(0, _C, edge, 0)
        if wait_sc:
            wait_scatter((ib - 1) % _NI, (g - 1) % _NG)
        if do_idx:
            issue_idx(k + 3, (ib + 3) % _NI)
        if do_g2:
            wait_idx(k + 2, (ib + 2) % _NI)
            issue_gather((ib + 2) % _NI, (g + 2) % _NG)
        issue_scatter(ib, g)

    issue_idx(0, 0)
    issue_idx(1, 1)
    issue_idx(2, 2)
    wait_idx(0, 0)
    issue_gather(0, 0)
    wait_idx(1, 1)
    issue_gather(1, 1)
    do_step(0, 0, 0, wait_sc=False)
    do_step(1, 1, 1)

    n_steady = (k_chunks - 5) // 12

    def block(j, carry):
        k0 = 12 * j + 2
        for i in range(12):
            do_step(k0 + i, (2 + i) % _NI, (2 + i) % _NG)
        return carry

    lax.fori_loop(0, n_steady, block, 0)

    for t in range(3):
        k = k_chunks - 3 + t
        do_step(k, k % _NI, k % _NG,
                do_idx=(k + 3 < k_chunks), do_g2=(k + 2 < k_chunks))

    wait_scatter((k_chunks - 1) % _NI, (k_chunks - 1) % _NG)

    plsc.subcore_barrier()

    pltpu.sync_copy(acc_sh.at[pl.ds(zoff, _ZROWS)],
                    out_hbm.at[pl.ds(c * n + zoff, _ZROWS)])


def _edge_aggregate(h, src, dst, adj):
    n, d = h.shape
    e = src.shape[0]
    zeros = jnp.zeros((_ZROWS, d), jnp.float32)
    mesh = plsc.VectorSubcoreMesh(core_axis_name="c", subcore_axis_name="s",
                                  num_cores=_NC, num_subcores=_NS)
    body = functools.partial(_edge_body, n, d, e)
    partials = pl.kernel(
        body,
        out_type=jax.ShapeDtypeStruct((_NC * n, d), jnp.float32),
        mesh=mesh,
        compiler_params=pltpu.CompilerParams(needs_layout_passes=False),
        scratch_types=[
            pltpu.VMEM((_NI, _C), jnp.int32),
            pltpu.VMEM((_NI, _C), jnp.int32),
            pltpu.VMEM((_NI, _C), jnp.float32),
            pltpu.VMEM((_NG, _C, d), jnp.float32),
            pltpu.VMEM_SHARED((n, d), jnp.float32),
        ] + [pltpu.SemaphoreType.DMA] * (_NI + 2 * _NG),
    )(h, src, dst, adj, zeros)
    return partials


def kernel(x, edge_index, adj_values, kernel):
    n = x.shape[0]
    h = _dense_transform(x, kernel)
    src = edge_index[0].astype(jnp.int32)
    dst = edge_index[1].astype(jnp.int32)
    partials = _edge_aggregate(h, src, dst, adj_values)
    return _combine_relu(partials)
